# Initial kernel scaffold; baseline (speedup 1.0000x reference)
import jax
import jax.numpy as jnp
from jax.experimental import pallas as pl


def kernel(imgPredict, imgLabel):
    raise NotImplementedError("write your pallas kernel here")


# speedup vs baseline: 1.3971x; 1.3971x over previous
import functools

import jax
import jax.numpy as jnp
from jax import lax
from jax.experimental import pallas as pl
from jax.experimental.pallas import tpu as pltpu
from jax.experimental.pallas import tpu_sc as plsc

NUM_CLASS = 19
P = 16 * 512 * 512
NC, NS, L = 2, 16, 16
NW = NC * NS
PER_W = P // NW
CHUNK = 16384
NCHUNK = PER_W // CHUNK
HBINS = 368
HSZ = HBINS * L

_mesh = plsc.VectorSubcoreMesh(core_axis_name="c", subcore_axis_name="s")


@functools.partial(
    pl.kernel,
    mesh=_mesh,
    compiler_params=pltpu.CompilerParams(needs_layout_passes=False),
    out_type=jax.ShapeDtypeStruct((NW, HSZ), jnp.int32),
    scratch_types=[
        pltpu.VMEM((2, CHUNK), jnp.int32),
        pltpu.VMEM((2, CHUNK), jnp.int32),
        pltpu.VMEM((HSZ,), jnp.int32),
        pltpu.SemaphoreType.DMA,
        pltpu.SemaphoreType.DMA,
    ],
)
def _hist_kernel(pred_hbm, label_hbm, out_hbm, pbuf, lbuf, hist, sem0, sem1):
    c = lax.axis_index("c")
    s = lax.axis_index("s")
    wid = c * NS + s
    base = wid * PER_W

    zeros = jnp.zeros((L,), jnp.int32)

    def zbody(i, carry):
        hist[pl.ds(i * L, L)] = zeros
        return carry

    lax.fori_loop(0, HBINS, zbody, 0)

    lane = lax.iota(jnp.int32, L)
    ones = jnp.ones((L,), jnp.int32)
    sems = (sem0, sem1)
    pending = [None, None]

    for g in range(NCHUNK + 1):
        if g < NCHUNK:
            slot = g % 2
            cp_p = pltpu.async_copy(
                pred_hbm.at[pl.ds(base + g * CHUNK, CHUNK)], pbuf.at[slot],
                sems[slot])
            cp_l = pltpu.async_copy(
                label_hbm.at[pl.ds(base + g * CHUNK, CHUNK)], lbuf.at[slot],
                sems[slot])
            pending[slot] = (cp_p, cp_l)
        if g >= 1:
            slot = (g - 1) % 2
            cp_p, cp_l = pending[slot]
            cp_p.wait()
            cp_l.wait()

            def vbody(i, carry):
                pv = pbuf[slot, pl.ds(i * L, L)]
                lv = lbuf[slot, pl.ds(i * L, L)]
                flat = (lv * NUM_CLASS + pv) * L + lane
                plsc.addupdate_scatter(hist, [flat], ones)
                return carry

            lax.fori_loop(0, CHUNK // L, vbody, 0)

    pltpu.sync_copy(hist, out_hbm.at[wid])


def kernel(imgPredict, imgLabel):
    pred = imgPredict.reshape(-1)
    label = imgLabel.reshape(-1)
    parts = _hist_kernel(pred, label)
    counts = parts.reshape(NW, HBINS, L).sum(axis=(0, 2))[: NUM_CLASS ** 2]
    cm = counts.reshape(NUM_CLASS, NUM_CLASS).astype(jnp.float32)

    diag = jnp.diag(cm)
    pa = diag.sum() / cm.sum()
    cpa = diag / cm.sum(axis=1)
    mpa = jnp.nanmean(cpa)
    union = cm.sum(axis=1) + cm.sum(axis=0) - diag
    ciou = diag / union
    miou = jnp.nanmean(ciou)
    return (pa, cpa, mpa, ciou, miou)


# speedup vs baseline: 2.3404x; 1.6752x over previous
import functools

import jax
import jax.numpy as jnp
from jax import lax
from jax.experimental import pallas as pl
from jax.experimental.pallas import tpu as pltpu
from jax.experimental.pallas import tpu_sc as plsc

NUM_CLASS = 19
P = 16 * 512 * 512
NC, NS, L = 2, 16, 16
NW = NC * NS
PER_W = P // NW
CHUNK = 16384
NCHUNK = PER_W // CHUNK
HBINS = 368
HSZ = HBINS * L

_mesh = plsc.VectorSubcoreMesh(core_axis_name="c", subcore_axis_name="s")


@functools.partial(
    pl.kernel,
    mesh=_mesh,
    compiler_params=pltpu.CompilerParams(needs_layout_passes=False),
    out_type=jax.ShapeDtypeStruct((NW, HSZ), jnp.int32),
    scratch_types=[
        pltpu.VMEM((2, CHUNK), jnp.int32),
        pltpu.VMEM((2, CHUNK), jnp.int32),
        pltpu.VMEM((HSZ,), jnp.int32),
        pltpu.SemaphoreType.DMA,
        pltpu.SemaphoreType.DMA,
    ],
)
def _hist_kernel(pred_hbm, label_hbm, out_hbm, pbuf, lbuf, hist, sem0, sem1):
    c = lax.axis_index("c")
    s = lax.axis_index("s")
    wid = c * NS + s
    base = wid * PER_W

    zeros = jnp.zeros((L,), jnp.int32)

    @plsc.parallel_loop(0, HBINS, unroll=8)
    def _zero(i):
        hist[pl.ds(i * L, L)] = zeros

    lane = lax.iota(jnp.int32, L)
    ones = jnp.ones((L,), jnp.int32)
    sems = (sem0, sem1)
    pending = [None, None]

    for g in range(NCHUNK + 1):
        if g < NCHUNK:
            slot = g % 2
            cp_p = pltpu.async_copy(
                pred_hbm.at[pl.ds(base + g * CHUNK, CHUNK)], pbuf.at[slot],
                sems[slot])
            cp_l = pltpu.async_copy(
                label_hbm.at[pl.ds(base + g * CHUNK, CHUNK)], lbuf.at[slot],
                sems[slot])
            pending[slot] = (cp_p, cp_l)
        if g >= 1:
            slot = (g - 1) % 2
            cp_p, cp_l = pending[slot]
            cp_p.wait()
            cp_l.wait()

            @plsc.parallel_loop(0, CHUNK // L, unroll=8)
            def _vbody(i):
                pv = pbuf[slot, pl.ds(i * L, L)]
                lv = lbuf[slot, pl.ds(i * L, L)]
                flat = (lv * NUM_CLASS + pv) * L + lane
                plsc.addupdate_scatter(hist, [flat], ones)

    pltpu.sync_copy(hist, out_hbm.at[wid])


def kernel(imgPredict, imgLabel):
    pred = imgPredict.reshape(-1)
    label = imgLabel.reshape(-1)
    parts = _hist_kernel(pred, label)
    counts = parts.reshape(NW, HBINS, L).sum(axis=(0, 2))[: NUM_CLASS ** 2]
    cm = counts.reshape(NUM_CLASS, NUM_CLASS).astype(jnp.float32)

    diag = jnp.diag(cm)
    pa = diag.sum() / cm.sum()
    cpa = diag / cm.sum(axis=1)
    mpa = jnp.nanmean(cpa)
    union = cm.sum(axis=1) + cm.sum(axis=0) - diag
    ciou = diag / union
    miou = jnp.nanmean(ciou)
    return (pa, cpa, mpa, ciou, miou)


# speedup vs baseline: 4.1069x; 1.7548x over previous
import functools

import jax
import jax.numpy as jnp
from jax import lax
from jax.experimental import pallas as pl
from jax.experimental.pallas import tpu as pltpu
from jax.experimental.pallas import tpu_sc as plsc

NUM_CLASS = 19
NIMG, H, W = 16, 512, 512
NC, NS, L = 2, 16, 16
NW = NC * NS
ROWS_W = H // 2
RCHUNK = 32
NCHUNK = ROWS_W // RCHUNK
VECS = RCHUNK * W // L
HBINS = 368
HSZ = HBINS * L

_mesh = plsc.VectorSubcoreMesh(core_axis_name="c", subcore_axis_name="s")


@functools.partial(
    pl.kernel,
    mesh=_mesh,
    compiler_params=pltpu.CompilerParams(
        needs_layout_passes=False, use_tc_tiling_on_sc=True),
    out_type=jax.ShapeDtypeStruct((NW, HSZ), jnp.int32),
    scratch_types=[
        pltpu.VMEM((2, RCHUNK, W), jnp.int32),
        pltpu.VMEM((2, RCHUNK, W), jnp.int32),
        pltpu.VMEM((HSZ,), jnp.int32),
        pltpu.SemaphoreType.DMA,
        pltpu.SemaphoreType.DMA,
    ],
)
def _hist_kernel(pred_hbm, label_hbm, out_hbm, pbuf, lbuf, hist, sem0, sem1):
    c = lax.axis_index("c")
    s = lax.axis_index("s")
    wid = c * NS + s
    img = wid // 2
    row0 = (wid % 2) * ROWS_W

    zeros = jnp.zeros((L,), jnp.int32)

    @plsc.parallel_loop(0, HBINS, unroll=8)
    def _zero(i):
        hist[pl.ds(i * L, L)] = zeros

    lane = lax.iota(jnp.int32, L)
    ones = jnp.ones((L,), jnp.int32)
    sems = (sem0, sem1)
    pending = [None, None]

    for g in range(NCHUNK + 1):
        if g < NCHUNK:
            slot = g % 2
            r = row0 + g * RCHUNK
            cp_p = pltpu.async_copy(
                pred_hbm.at[img, pl.ds(r, RCHUNK), :], pbuf.at[slot],
                sems[slot])
            cp_l = pltpu.async_copy(
                label_hbm.at[img, pl.ds(r, RCHUNK), :], lbuf.at[slot],
                sems[slot])
            pending[slot] = (cp_p, cp_l)
        if g >= 1:
            slot = (g - 1) % 2
            cp_p, cp_l = pending[slot]
            cp_p.wait()
            cp_l.wait()

            @plsc.parallel_loop(0, VECS, unroll=8)
            def _vbody(i):
                row = i >> 5
                col = (i & 31) * L
                pv = pbuf[slot, row, pl.ds(col, L)]
                lv = lbuf[slot, row, pl.ds(col, L)]
                flat = (lv * NUM_CLASS + pv) * L + lane
                plsc.addupdate_scatter(hist, [flat], ones)

    pltpu.sync_copy(hist, out_hbm.at[wid])


def kernel(imgPredict, imgLabel):
    parts = _hist_kernel(imgPredict, imgLabel)
    counts = parts.reshape(NW, HBINS, L).sum(axis=(0, 2))[: NUM_CLASS ** 2]
    cm = counts.reshape(NUM_CLASS, NUM_CLASS).astype(jnp.float32)

    eye = jnp.eye(NUM_CLASS, dtype=jnp.float32)
    diag = (cm * eye).sum(axis=1)
    pa = diag.sum() / cm.sum()
    cpa = diag / cm.sum(axis=1)
    mpa = jnp.nanmean(cpa)
    union = cm.sum(axis=1) + cm.sum(axis=0) - diag
    ciou = diag / union
    miou = jnp.nanmean(ciou)
    return (pa, cpa, mpa, ciou, miou)
